# baseline (speedup 1.0000x reference)
import functools

import jax
import jax.numpy as jnp
from jax.experimental import pallas as pl
from jax.experimental.pallas import tpu as pltpu

_GSHARD_W = 0.0
_IMPORTANCE_W = 1.0


def _tile_body(x_ref, w_ref, gates_ref, aux_ref, gshard_ref, imp_ref,
               imp_acc, gsh_acc,
               *, num_batch, num_experts, num_tiles, seq_len):
    i = pl.program_id(0)
    w = w_ref[...]
    oh_sum = None
    g_sum = None
    for b in range(num_batch):
        x = x_ref[b]
        logits = jax.lax.dot_general(
            x, w, (((1,), (0,)), ((), ())),
            preferred_element_type=jnp.float32)
        m = jnp.max(logits, axis=1, keepdims=True)
        e = jnp.exp(logits - m)
        s = jnp.sum(e, axis=1, keepdims=True)
        gates = e / s
        gates_ref[b] = gates
        lane = jax.lax.broadcasted_iota(jnp.int32, logits.shape, 1)
        eq = logits == m
        amin = jnp.min(jnp.where(eq, lane, num_experts), axis=1,
                       keepdims=True)
        onehot = (lane == amin).astype(jnp.float32)
        oh_sum = onehot if oh_sum is None else oh_sum + onehot
        g_sum = gates if g_sum is None else g_sum + gates

    imp_tile = jnp.sum(g_sum, axis=0, keepdims=True)
    gsh_tile = jnp.sum(oh_sum * g_sum)

    @pl.when(i == 0)
    def _init():
        imp_acc[...] = imp_tile
        gsh_acc[0, 0] = gsh_tile

    @pl.when(i != 0)
    def _accum():
        imp_acc[...] += imp_tile
        gsh_acc[0, 0] += gsh_tile

    @pl.when(i == num_tiles - 1)
    def _finalize():
        imp = imp_acc[...]
        mean = jnp.sum(imp) / num_experts
        var = jnp.sum((imp - mean) ** 2) / (num_experts - 1)
        imp_loss = var / (mean * mean)
        gshard = gsh_acc[0, 0] * (
            num_experts / (seq_len * num_batch * num_batch))
        total_w = _GSHARD_W + _IMPORTANCE_W
        aux_loss = (_GSHARD_W * gshard + _IMPORTANCE_W * imp_loss) / total_w
        imp_ref[...] = jnp.reshape(imp_loss, (1, 1))
        gshard_ref[...] = jnp.reshape(gshard, (1, 1))
        aux_ref[...] = jnp.reshape(aux_loss, (1, 1))


@functools.partial(jax.jit, static_argnames=("tile_s",))
def _router(x, W, tile_s=128):
    B, S, H = x.shape
    E = W.shape[0]
    num_tiles = S // tile_s

    tile_body = functools.partial(
        _tile_body, num_batch=B, num_experts=E, num_tiles=num_tiles,
        seq_len=S)
    scalar_shape = jax.ShapeDtypeStruct((1, 1), jnp.float32)
    scalar_spec = pl.BlockSpec((1, 1), lambda i: (0, 0))
    gates, aux, gshard, imp = pl.pallas_call(
        tile_body,
        grid=(num_tiles,),
        in_specs=[
            pl.BlockSpec((B, tile_s, H), lambda i: (0, i, 0)),
            pl.BlockSpec((H, E), lambda i: (0, 0)),
        ],
        out_specs=(
            pl.BlockSpec((B, tile_s, E), lambda i: (0, i, 0)),
            scalar_spec, scalar_spec, scalar_spec,
        ),
        out_shape=(
            jax.ShapeDtypeStruct((B, S, E), jnp.float32),
            scalar_shape, scalar_shape, scalar_shape,
        ),
        scratch_shapes=[
            pltpu.VMEM((1, E), jnp.float32),
            pltpu.SMEM((1, 1), jnp.float32),
        ],
        compiler_params=pltpu.CompilerParams(
            dimension_semantics=("arbitrary",)),
    )(x, W.T)

    return gates, aux.reshape(()), gshard.reshape(()), imp.reshape(())


def kernel(x, W):
    return _router(x, W)


# speedup vs baseline: 1.1562x; 1.1562x over previous
import functools

import jax
import jax.numpy as jnp
from jax.experimental import pallas as pl
from jax.experimental.pallas import tpu as pltpu

_GSHARD_W = 0.0
_IMPORTANCE_W = 1.0


def _tile_body(x_ref, w_ref, gates_ref, aux_ref, gshard_ref, imp_ref,
               imp_acc, gsh_acc,
               *, num_batch, num_experts, num_tiles, seq_len):
    i = pl.program_id(0)
    w = w_ref[...]
    oh_sum = None
    g_sum = None
    for b in range(num_batch):
        x = x_ref[b]
        logits = jax.lax.dot_general(
            x, w, (((1,), (0,)), ((), ())),
            preferred_element_type=jnp.float32)
        m = jnp.max(logits, axis=1, keepdims=True)
        e = jnp.exp(logits - m)
        s = jnp.sum(e, axis=1, keepdims=True)
        gates = e / s
        gates_ref[b] = gates
        lane = jax.lax.broadcasted_iota(jnp.int32, logits.shape, 1)
        eq = logits == m
        amin = jnp.min(jnp.where(eq, lane, num_experts), axis=1,
                       keepdims=True)
        onehot = (lane == amin).astype(jnp.float32)
        oh_sum = onehot if oh_sum is None else oh_sum + onehot
        g_sum = gates if g_sum is None else g_sum + gates

    imp_tile = jnp.sum(g_sum, axis=0, keepdims=True)
    gsh_tile = jnp.sum(oh_sum * g_sum)

    @pl.when(i == 0)
    def _init():
        imp_acc[...] = imp_tile
        gsh_acc[0, 0] = gsh_tile

    @pl.when(i != 0)
    def _accum():
        imp_acc[...] += imp_tile
        gsh_acc[0, 0] += gsh_tile

    @pl.when(i == num_tiles - 1)
    def _finalize():
        imp = imp_acc[...]
        mean = jnp.sum(imp) / num_experts
        var = jnp.sum((imp - mean) ** 2) / (num_experts - 1)
        imp_loss = var / (mean * mean)
        gshard = gsh_acc[0, 0] * (
            num_experts / (seq_len * num_batch * num_batch))
        total_w = _GSHARD_W + _IMPORTANCE_W
        aux_loss = (_GSHARD_W * gshard + _IMPORTANCE_W * imp_loss) / total_w
        imp_ref[...] = jnp.reshape(imp_loss, (1, 1))
        gshard_ref[...] = jnp.reshape(gshard, (1, 1))
        aux_ref[...] = jnp.reshape(aux_loss, (1, 1))


@functools.partial(jax.jit, static_argnames=("tile_s",))
def _router(x, W, tile_s=256):
    B, S, H = x.shape
    E = W.shape[0]
    num_tiles = S // tile_s

    tile_body = functools.partial(
        _tile_body, num_batch=B, num_experts=E, num_tiles=num_tiles,
        seq_len=S)
    scalar_shape = jax.ShapeDtypeStruct((1, 1), jnp.float32)
    scalar_spec = pl.BlockSpec((1, 1), lambda i: (0, 0))
    gates, aux, gshard, imp = pl.pallas_call(
        tile_body,
        grid=(num_tiles,),
        in_specs=[
            pl.BlockSpec((B, tile_s, H), lambda i: (0, i, 0)),
            pl.BlockSpec((H, E), lambda i: (0, 0)),
        ],
        out_specs=(
            pl.BlockSpec((B, tile_s, E), lambda i: (0, i, 0)),
            scalar_spec, scalar_spec, scalar_spec,
        ),
        out_shape=(
            jax.ShapeDtypeStruct((B, S, E), jnp.float32),
            scalar_shape, scalar_shape, scalar_shape,
        ),
        scratch_shapes=[
            pltpu.VMEM((1, E), jnp.float32),
            pltpu.SMEM((1, 1), jnp.float32),
        ],
        compiler_params=pltpu.CompilerParams(
            dimension_semantics=("arbitrary",)),
    )(x, W.T)

    return gates, aux.reshape(()), gshard.reshape(()), imp.reshape(())


def kernel(x, W):
    return _router(x, W)
